# Initial kernel scaffold; baseline (speedup 1.0000x reference)
import jax
import jax.numpy as jnp
from jax.experimental import pallas as pl


def kernel(x, weight):
    raise NotImplementedError("write your pallas kernel here")


# speedup vs baseline: 1.6343x; 1.6343x over previous
import functools

import jax
import jax.numpy as jnp
from jax import lax
from jax.experimental import pallas as pl
from jax.experimental.pallas import tpu as pltpu
from jax.experimental.pallas import tpu_sc as plsc

_B, _C, _H, _W = 8, 256, 32, 32
_N = _H * _W
_M = _B * _N
_K = 8192

_BM = 1024
_BK = 2048
_MB = _M // _BM
_KB = _K // _BK

_NW = 32
_BPW = _M // _NW


def _argmin_body(x_ref, wt_ref, xsq_ref, esq_ref, idx_ref, loss_ref,
                 run_min, run_idx):
    m = pl.program_id(0)
    k = pl.program_id(1)

    @pl.when(jnp.logical_and(m == 0, k == 0))
    def _init_loss():
        loss_ref[...] = jnp.zeros_like(loss_ref)

    @pl.when(k == 0)
    def _init_running():
        run_min[...] = jnp.full_like(run_min, jnp.inf)
        run_idx[...] = jnp.zeros_like(run_idx)

    xb = x_ref[...]
    wtb = wt_ref[...]
    cross = jnp.dot(xb, wtb, preferred_element_type=jnp.float32)
    xsq = xsq_ref[...]
    esq = esq_ref[...].reshape(1, _BK)
    d2 = (xsq + esq) - 2.0 * cross
    d2 = jnp.maximum(d2, 0.0)

    mn = jnp.min(d2, axis=1, keepdims=True)
    col = lax.broadcasted_iota(jnp.int32, d2.shape, 1)
    arg = jnp.min(jnp.where(d2 == mn, col, _BK), axis=1, keepdims=True)

    cur = run_min[...]
    better = mn < cur
    run_min[...] = jnp.where(better, mn, cur)
    run_idx[...] = jnp.where(better, arg + k * _BK, run_idx[...])

    @pl.when(k == _KB - 1)
    def _finish():
        idx_ref[...] = run_idx[...]
        loss_ref[...] = loss_ref[...] + jnp.sum(run_min[...], keepdims=True).reshape(1, 1)


_argmin_call = pl.pallas_call(
    _argmin_body,
    grid=(_MB, _KB),
    in_specs=[
        pl.BlockSpec((_BM, _C), lambda m, k: (m, 0)),
        pl.BlockSpec((_C, _BK), lambda m, k: (0, k)),
        pl.BlockSpec((_BM, 1), lambda m, k: (m, 0)),
        pl.BlockSpec((1, 1, _BK), lambda m, k: (k, 0, 0)),
    ],
    out_specs=[
        pl.BlockSpec((_BM, 1), lambda m, k: (m, 0)),
        pl.BlockSpec((1, 1), lambda m, k: (0, 0)),
    ],
    out_shape=[
        jax.ShapeDtypeStruct((_M, 1), jnp.int32),
        jax.ShapeDtypeStruct((1, 1), jnp.float32),
    ],
    scratch_shapes=[
        pltpu.VMEM((_BM, 1), jnp.float32),
        pltpu.VMEM((_BM, 1), jnp.int32),
    ],
)


@functools.cache
def _make_gather_rows():
    mesh = plsc.VectorSubcoreMesh(core_axis_name="c", subcore_axis_name="s")

    @functools.partial(
        pl.kernel,
        mesh=mesh,
        out_type=jax.ShapeDtypeStruct((_M, _C), jnp.float32),
        scratch_types=[
            pltpu.VMEM((_BPW,), jnp.int32),
            pltpu.VMEM((_BPW, _C), jnp.float32),
            pltpu.SemaphoreType.DMA,
        ],
    )
    def _gather_rows(table_hbm, idx_hbm, out_hbm, idx_v, rows_v, sem):
        wid = lax.axis_index("s") * 2 + lax.axis_index("c")
        base = wid * _BPW
        pltpu.sync_copy(idx_hbm.at[pl.ds(base, _BPW)], idx_v)
        pltpu.async_copy(table_hbm.at[idx_v], rows_v, sem).wait()
        pltpu.sync_copy(rows_v, out_hbm.at[pl.ds(base, _BPW)])

    return _gather_rows


def kernel(x, weight):
    Bs, Cs, Hs, Ws = x.shape
    x_flat = jnp.transpose(x, (0, 2, 3, 1)).reshape(Bs * Hs * Ws, Cs)
    x_sq = jnp.sum(x_flat ** 2, axis=-1, keepdims=True)
    e_sq = jnp.sum(weight ** 2, axis=-1)
    wt = weight.T

    idx2, loss_sum = _argmin_call(x_flat, wt, x_sq, e_sq.reshape(_KB, 1, _BK))
    idx = idx2.reshape(-1)

    quant = _make_gather_rows()(weight, idx)

    x_out = jnp.transpose(quant.reshape(Bs, Hs, Ws, Cs), (0, 3, 1, 2))
    loss = loss_sum[0, 0] / jnp.float32(_M * _C)
    idx_out = idx.reshape(Bs, Hs, Ws)
    return (x_out, loss, loss, idx_out)


# speedup vs baseline: 1.6351x; 1.0005x over previous
import functools

import jax
import jax.numpy as jnp
from jax import lax
from jax.experimental import pallas as pl
from jax.experimental.pallas import tpu as pltpu
from jax.experimental.pallas import tpu_sc as plsc

_B, _C, _H, _W = 8, 256, 32, 32
_N = _H * _W
_M = _B * _N
_K = 8192

_BM = 1024
_BK = 2048
_MB = _M // _BM
_KB = _K // _BK

_NW = 32
_BPW = _M // _NW


def _argmin_body(x_ref, wt_ref, xsq_ref, esq_ref, idx_ref, loss_ref,
                 run_min, run_idx, colf_ref):
    m = pl.program_id(0)
    k = pl.program_id(1)

    @pl.when(jnp.logical_and(m == 0, k == 0))
    def _init_loss():
        loss_ref[...] = jnp.zeros_like(loss_ref)
        colf_ref[...] = lax.broadcasted_iota(
            jnp.int32, colf_ref.shape, 1).astype(jnp.float32)

    @pl.when(k == 0)
    def _init_running():
        run_min[...] = jnp.full_like(run_min, jnp.inf)
        run_idx[...] = jnp.zeros_like(run_idx)

    xb = x_ref[...]
    w2tb = wt_ref[...]
    cross2 = jnp.dot(xb, w2tb, preferred_element_type=jnp.float32)
    xsq = xsq_ref[...]
    esq = esq_ref[...].reshape(1, _BK)
    d2 = (xsq + esq) - cross2

    mn = jnp.min(d2, axis=1, keepdims=True)
    arg = jnp.min(jnp.where(d2 == mn, colf_ref[...], jnp.float32(_BK)),
                  axis=1, keepdims=True)

    cur = run_min[...]
    better = mn < cur
    run_min[...] = jnp.where(better, mn, cur)
    run_idx[...] = jnp.where(better, arg + (k * _BK).astype(jnp.float32),
                             run_idx[...])

    @pl.when(k == _KB - 1)
    def _finish():
        idx_ref[...] = run_idx[...].astype(jnp.int32)
        loss_ref[...] = loss_ref[...] + jnp.sum(run_min[...], keepdims=True).reshape(1, 1)


_argmin_call = pl.pallas_call(
    _argmin_body,
    grid=(_MB, _KB),
    in_specs=[
        pl.BlockSpec((_BM, _C), lambda m, k: (m, 0)),
        pl.BlockSpec((_C, _BK), lambda m, k: (0, k)),
        pl.BlockSpec((_BM, 1), lambda m, k: (m, 0)),
        pl.BlockSpec((1, 1, _BK), lambda m, k: (k, 0, 0)),
    ],
    out_specs=[
        pl.BlockSpec((_BM, 1), lambda m, k: (m, 0)),
        pl.BlockSpec((1, 1), lambda m, k: (0, 0)),
    ],
    out_shape=[
        jax.ShapeDtypeStruct((_M, 1), jnp.int32),
        jax.ShapeDtypeStruct((1, 1), jnp.float32),
    ],
    scratch_shapes=[
        pltpu.VMEM((_BM, 1), jnp.float32),
        pltpu.VMEM((_BM, 1), jnp.float32),
        pltpu.VMEM((_BM, _BK), jnp.float32),
    ],
)


@functools.cache
def _make_gather_rows():
    mesh = plsc.VectorSubcoreMesh(core_axis_name="c", subcore_axis_name="s")

    @functools.partial(
        pl.kernel,
        mesh=mesh,
        out_type=jax.ShapeDtypeStruct((_M, _C), jnp.float32),
        scratch_types=[
            pltpu.VMEM((_BPW,), jnp.int32),
            pltpu.VMEM((_BPW, _C), jnp.float32),
            pltpu.SemaphoreType.DMA,
        ],
    )
    def _gather_rows(table_hbm, idx_hbm, out_hbm, idx_v, rows_v, sem):
        wid = lax.axis_index("s") * 2 + lax.axis_index("c")
        base = wid * _BPW
        pltpu.sync_copy(idx_hbm.at[pl.ds(base, _BPW)], idx_v)
        pltpu.async_copy(table_hbm.at[idx_v], rows_v, sem).wait()
        pltpu.sync_copy(rows_v, out_hbm.at[pl.ds(base, _BPW)])

    return _gather_rows


def kernel(x, weight):
    Bs, Cs, Hs, Ws = x.shape
    x_flat = jnp.transpose(x, (0, 2, 3, 1)).reshape(Bs * Hs * Ws, Cs)
    x_sq = jnp.sum(x_flat ** 2, axis=-1, keepdims=True)
    e_sq = jnp.sum(weight ** 2, axis=-1)
    w2t = 2.0 * weight.T

    idx2, loss_sum = _argmin_call(x_flat, w2t, x_sq, e_sq.reshape(_KB, 1, _BK))
    idx = idx2.reshape(-1)

    quant = _make_gather_rows()(weight, idx)

    x_out = jnp.transpose(quant.reshape(Bs, Hs, Ws, Cs), (0, 3, 1, 2))
    loss = loss_sum[0, 0] / jnp.float32(_M * _C)
    idx_out = idx.reshape(Bs, Hs, Ws)
    return (x_out, loss, loss, idx_out)


# speedup vs baseline: 1.6948x; 1.0365x over previous
import functools

import jax
import jax.numpy as jnp
from jax import lax
from jax.experimental import pallas as pl
from jax.experimental.pallas import tpu as pltpu
from jax.experimental.pallas import tpu_sc as plsc

_B, _C, _H, _W = 8, 256, 32, 32
_N = _H * _W
_M = _B * _N
_K = 8192

_BM = 1024
_BK = 2048
_MB = _M // _BM
_KB = _K // _BK

_NW = 32
_BPW = _M // _NW


def _argmin_body(x_ref, wt_ref, xsq_ref, esq_ref, idx_ref, loss_ref,
                 run_min, run_idx, colf_ref):
    m = pl.program_id(0)
    k = pl.program_id(1)

    @pl.when(jnp.logical_and(m == 0, k == 0))
    def _init_loss():
        loss_ref[...] = jnp.zeros_like(loss_ref)
        colf_ref[...] = lax.broadcasted_iota(
            jnp.int32, colf_ref.shape, 1).astype(jnp.float32)

    @pl.when(k == 0)
    def _init_running():
        run_min[...] = jnp.full_like(run_min, jnp.inf)
        run_idx[...] = jnp.zeros_like(run_idx)

    xb = x_ref[...]
    w2b = wt_ref[...]
    cross2 = lax.dot_general(
        xb, w2b, (((1,), (1,)), ((), ())),
        preferred_element_type=jnp.float32)
    xsq = xsq_ref[...]
    esq = esq_ref[...].reshape(1, _BK)
    d2 = (xsq + esq) - cross2

    mn = jnp.min(d2, axis=1, keepdims=True)
    arg = jnp.min(jnp.where(d2 == mn, colf_ref[...], jnp.float32(_BK)),
                  axis=1, keepdims=True)

    cur = run_min[...]
    better = mn < cur
    run_min[...] = jnp.where(better, mn, cur)
    run_idx[...] = jnp.where(better, arg + (k * _BK).astype(jnp.float32),
                             run_idx[...])

    @pl.when(k == _KB - 1)
    def _finish():
        idx_ref[...] = run_idx[...].astype(jnp.int32)
        loss_ref[...] = loss_ref[...] + jnp.sum(run_min[...], keepdims=True).reshape(1, 1)


_argmin_call = pl.pallas_call(
    _argmin_body,
    grid=(_MB, _KB),
    in_specs=[
        pl.BlockSpec((_BM, _C), lambda m, k: (m, 0)),
        pl.BlockSpec((_BK, _C), lambda m, k: (k, 0)),
        pl.BlockSpec((_BM, 1), lambda m, k: (m, 0)),
        pl.BlockSpec((1, 1, _BK), lambda m, k: (k, 0, 0)),
    ],
    out_specs=[
        pl.BlockSpec((_BM, 1), lambda m, k: (m, 0)),
        pl.BlockSpec((1, 1), lambda m, k: (0, 0)),
    ],
    out_shape=[
        jax.ShapeDtypeStruct((_M, 1), jnp.int32),
        jax.ShapeDtypeStruct((1, 1), jnp.float32),
    ],
    scratch_shapes=[
        pltpu.VMEM((_BM, 1), jnp.float32),
        pltpu.VMEM((_BM, 1), jnp.float32),
        pltpu.VMEM((_BM, _BK), jnp.float32),
    ],
)


@functools.cache
def _make_gather_rows():
    mesh = plsc.VectorSubcoreMesh(core_axis_name="c", subcore_axis_name="s")

    @functools.partial(
        pl.kernel,
        mesh=mesh,
        out_type=jax.ShapeDtypeStruct((_M, _C), jnp.float32),
        scratch_types=[
            pltpu.VMEM((_BPW,), jnp.int32),
            pltpu.VMEM((_BPW, _C), jnp.float32),
            pltpu.SemaphoreType.DMA,
        ],
    )
    def _gather_rows(table_hbm, idx_hbm, out_hbm, idx_v, rows_v, sem):
        wid = lax.axis_index("s") * 2 + lax.axis_index("c")
        base = wid * _BPW
        pltpu.sync_copy(idx_hbm.at[pl.ds(base, _BPW)], idx_v)
        pltpu.async_copy(table_hbm.at[idx_v], rows_v, sem).wait()
        pltpu.sync_copy(rows_v, out_hbm.at[pl.ds(base, _BPW)])

    return _gather_rows


def kernel(x, weight):
    Bs, Cs, Hs, Ws = x.shape
    x_flat = jnp.transpose(x, (0, 2, 3, 1)).reshape(Bs * Hs * Ws, Cs)
    x_sq = jnp.sum(x_flat ** 2, axis=-1, keepdims=True)
    e_sq = jnp.sum(weight ** 2, axis=-1)
    w2 = 2.0 * weight

    idx2, loss_sum = _argmin_call(x_flat, w2, x_sq, e_sq.reshape(_KB, 1, _BK))
    idx = idx2.reshape(-1)

    quant = _make_gather_rows()(weight, idx)

    x_out = jnp.transpose(quant.reshape(Bs, Hs, Ws, Cs), (0, 3, 1, 2))
    loss = loss_sum[0, 0] / jnp.float32(_M * _C)
    idx_out = idx.reshape(Bs, Hs, Ws)
    return (x_out, loss, loss, idx_out)
